# baseline (speedup 1.0000x reference)
import jax
import jax.numpy as jnp
from jax import lax
from jax.experimental import pallas as pl
from jax.experimental.pallas import tpu as pltpu
from jax.experimental.pallas import tpu_sc as plsc

_info = plsc.get_sparse_core_info()
_NC = _info.num_cores
_NS = _info.num_subcores
_NW = _NC * _NS

_H = 50
_B = 16384
_D = 32
_BW = _B // _NW
_TB = _BW // 128
_NU = _H * _TB


def _emb_body(ids_t, tpack, out_t, ids_buf, q_buf, r_buf, fet, out_loc,
              sem_g, sem_w):
    wid = lax.axis_index("s") * _NC + lax.axis_index("c")
    b0 = wid * _BW

    pltpu.sync_copy(ids_t.at[:, pl.ds(b0, _BW)], ids_buf)

    iota = lax.iota(jnp.int32, 16)

    rows = [iota + (16 * v) for v in range(8)]

    def qr(h, i):
        for v in range(8):
            idx = ids_buf[h, pl.ds(i * 128 + 16 * v, 16)]
            q_buf[i, pl.ds(16 * v, 16)] = lax.shift_right_logical(idx, 2)
            r_buf[i, pl.ds(16 * v, 16)] = lax.bitwise_and(idx, 3)

    def gather_start(i):
        pltpu.async_copy(tpack.at[q_buf.at[i]], fet.at[i], sem_g.at[i])

    def gather_wait(i):
        pltpu.make_async_copy(tpack.at[pl.ds(0, 128)], fet.at[i],
                              sem_g.at[i]).wait()

    def write_start(h, i):
        pltpu.async_copy(out_loc.at[i % 2],
                         out_t.at[h, :, pl.ds(b0 + i * 128, 128)],
                         sem_w.at[i % 2])

    def write_wait(i):
        pltpu.make_async_copy(out_loc.at[i % 2],
                              out_t.at[0, :, pl.ds(0, 128)],
                              sem_w.at[i % 2]).wait()

    def extract(i):
        rv = [r_buf[i, pl.ds(16 * v, 16)] * 32 for v in range(8)]
        for j in range(_D):
            for v in range(8):
                vals = plsc.load_gather(fet.at[i], [rows[v], rv[v] + j])
                out_loc[i % 2, j, pl.ds(16 * v, 16)] = vals

    for i in range(4):
        qr(0, i)
        gather_start(i)

    def group(g, carry):
        for i in range(4):
            if i >= 2:
                write_wait(i)
            else:
                @pl.when(g >= 1)
                def _():
                    write_wait(i)
            gather_wait(i)
            extract(i)

            @pl.when(g < _H - 1)
            def _():
                qr(g + 1, i)
                gather_start(i)

            write_start(g, i)
        return carry

    lax.fori_loop(0, _H, group, 0)
    write_wait(0)
    write_wait(1)


def _emb(ids_t, tpack):
    return pl.kernel(
        _emb_body,
        out_type=jax.ShapeDtypeStruct((_H, _D, _B), jnp.float32),
        mesh=plsc.VectorSubcoreMesh(core_axis_name="c", subcore_axis_name="s"),
        scratch_types=[
            pltpu.VMEM((_H, _BW), jnp.int32),
            pltpu.VMEM((4, 128), jnp.int32),
            pltpu.VMEM((4, 128), jnp.int32),
            pltpu.VMEM((4, 128, 128), jnp.float32),
            pltpu.VMEM((2, _D, 128), jnp.float32),
            pltpu.SemaphoreType.DMA((4,)),
            pltpu.SemaphoreType.DMA((2,)),
        ],
        compiler_params=pltpu.CompilerParams(needs_layout_passes=False),
    )(ids_t, tpack)


def kernel(input_ids, embedding_matrix):
    ids_t = input_ids.T
    tpack = embedding_matrix.reshape(-1, 128)
    out_t = _emb(ids_t.astype(jnp.int32), tpack)
    return out_t.transpose(2, 0, 1)


# speedup vs baseline: 1.1988x; 1.1988x over previous
import jax
import jax.numpy as jnp
from jax import lax
from jax.experimental import pallas as pl
from jax.experimental.pallas import tpu as pltpu
from jax.experimental.pallas import tpu_sc as plsc

_info = plsc.get_sparse_core_info()
_NC = _info.num_cores
_NS = _info.num_subcores
_NW = _NC * _NS

_H = 50
_B = 16384
_D = 32
_BW = _B // _NW
_TB = _BW // 128
_NU = _H * _TB


def _emb_body(ids_t, tpack, out_t, ids_buf, q_buf, r_buf, fet, out_loc,
              sem_g, sem_w):
    wid = lax.axis_index("s") * _NC + lax.axis_index("c")
    b0 = wid * _BW

    pltpu.sync_copy(ids_t.at[:, pl.ds(b0, _BW)], ids_buf)

    iota = lax.iota(jnp.int32, 16)

    rows = [iota + (16 * v) for v in range(8)]

    def qr(h, i):
        for v in range(8):
            idx = ids_buf[h, pl.ds(i * 128 + 16 * v, 16)]
            q_buf[i, pl.ds(16 * v, 16)] = lax.shift_right_logical(idx, 2)
            r_buf[i, pl.ds(16 * v, 16)] = lax.bitwise_and(idx, 3)

    def gather_start(i):
        pltpu.async_copy(tpack.at[q_buf.at[i]], fet.at[i], sem_g.at[i])

    def gather_wait(i):
        pltpu.make_async_copy(tpack.at[pl.ds(0, 128)], fet.at[i],
                              sem_g.at[i]).wait()

    def write_start(h, i):
        pltpu.async_copy(out_loc.at[i % 2],
                         out_t.at[h, :, pl.ds(b0 + i * 128, 128)],
                         sem_w.at[i % 2])

    def write_wait(i):
        pltpu.make_async_copy(out_loc.at[i % 2],
                              out_t.at[0, :, pl.ds(0, 128)],
                              sem_w.at[i % 2]).wait()

    def extract(i):
        rv = [r_buf[i, pl.ds(16 * v, 16)] * 32 for v in range(8)]
        for j in range(_D):
            rj = lax.bitwise_and(iota + j, 31)
            for v in range(8):
                vals = plsc.load_gather(fet.at[i], [rows[v], rv[v] + rj])
                plsc.store_scatter(out_loc.at[i % 2], [rj, rows[v]], vals)

    for i in range(4):
        qr(0, i)
        gather_start(i)

    def group(g, carry):
        for i in range(4):
            if i >= 2:
                write_wait(i)
            else:
                @pl.when(g >= 1)
                def _():
                    write_wait(i)
            gather_wait(i)
            extract(i)

            @pl.when(g < _H - 1)
            def _():
                qr(g + 1, i)
                gather_start(i)

            write_start(g, i)
        return carry

    lax.fori_loop(0, _H, group, 0)
    write_wait(0)
    write_wait(1)


def _emb(ids_t, tpack):
    return pl.kernel(
        _emb_body,
        out_type=jax.ShapeDtypeStruct((_H, _D, _B), jnp.float32),
        mesh=plsc.VectorSubcoreMesh(core_axis_name="c", subcore_axis_name="s"),
        scratch_types=[
            pltpu.VMEM((_H, _BW), jnp.int32),
            pltpu.VMEM((4, 128), jnp.int32),
            pltpu.VMEM((4, 128), jnp.int32),
            pltpu.VMEM((4, 128, 128), jnp.float32),
            pltpu.VMEM((2, _D, 128), jnp.float32),
            pltpu.SemaphoreType.DMA((4,)),
            pltpu.SemaphoreType.DMA((2,)),
        ],
        compiler_params=pltpu.CompilerParams(needs_layout_passes=False),
    )(ids_t, tpack)


def kernel(input_ids, embedding_matrix):
    ids_t = input_ids.T
    tpack = embedding_matrix.reshape(-1, 128)
    out_t = _emb(ids_t.astype(jnp.int32), tpack)
    return out_t.transpose(2, 0, 1)
